# baseline (speedup 1.0000x reference)
import functools

import jax
import jax.numpy as jnp
from jax import lax
from jax.experimental import pallas as pl
from jax.experimental.pallas import tpu as pltpu
from jax.experimental.pallas import tpu_sc as plsc

BATCH = 16384
HIST = 200
HIDDEN = 64
TABLE_ROWS = 1000

NC, NS = 2, 16
NW = NC * NS
B = BATCH * HIST
IDX_PER_W = B // NW
CH = 512
NCHUNK = IDX_PER_W // CH
NROWS = B // CH
GATHER_SPLITS = [(0, 128), (128, 128), (256, 128), (384, 128)]

_mesh = plsc.VectorSubcoreMesh(core_axis_name="c", subcore_axis_name="s")


@functools.partial(
    pl.kernel,
    out_type=jax.ShapeDtypeStruct((B, HIDDEN), jnp.float32),
    mesh=_mesh,
    scratch_types=[
        pltpu.VMEM((4, 1, CH), jnp.int32),
        pltpu.VMEM((2, CH, HIDDEN), jnp.float32),
        pltpu.VMEM_SHARED((TABLE_ROWS, HIDDEN), jnp.float32),
        pltpu.SemaphoreType.DMA,
        pltpu.SemaphoreType.DMA,
        pltpu.SemaphoreType.DMA,
        pltpu.SemaphoreType.DMA,
        pltpu.SemaphoreType.DMA,
        pltpu.SemaphoreType.DMA,
        pltpu.SemaphoreType.DMA,
        pltpu.SemaphoreType.DMA,
    ],
    compiler_params=pltpu.CompilerParams(use_tc_tiling_on_sc=False),
)
def _emb_lookup(idx_hbm, table_hbm, out_hbm, idx_v, rows_v, table_s,
                is0, is1, is2, is3, gs0, gs1, os0, os1):
  isems = (is0, is1, is2, is3)
  gsems = (gs0, gs1)
  osems = (os0, os1)
  sid = lax.axis_index("s")
  wid = sid * NC + lax.axis_index("c")
  wbase = wid * IDX_PER_W
  wrow = wid * NCHUNK

  @pl.when(sid == 0)
  def _():
    pltpu.sync_copy(table_hbm, table_s)

  plsc.subcore_barrier()

  def idx_copy(i, q):
    return pltpu.make_async_copy(
        idx_hbm.at[pl.ds(wrow + i, 1)], idx_v.at[q], isems[q])

  def gather_copies(i, q, b):
    src = table_s if b == 0 else table_hbm
    return [
        pltpu.make_async_copy(
            src.at[idx_v.at[q, 0, pl.ds(o, n)]],
            rows_v.at[b, pl.ds(o, n)], gsems[b])
        for o, n in GATHER_SPLITS
    ]

  def out_copy(i, b):
    return pltpu.make_async_copy(
        rows_v.at[b],
        out_hbm.at[pl.ds(wbase + i * CH, CH)], osems[b])

  idx_copy(0, 0).start()
  idx_copy(1, 1).start()

  idx_copy(0, 0).wait()
  for cp in gather_copies(0, 0, 0):
    cp.start()
  idx_copy(2, 2).start()

  idx_copy(1, 1).wait()
  for cp in gather_copies(1, 1, 1):
    cp.start()
  idx_copy(3, 3).start()
  for cp in gather_copies(0, 0, 0):
    cp.wait()
  out_copy(0, 0).start()

  def step(i, q, b, prefetch):
    idx_copy(i, q).wait()
    out_copy(i - 2, b).wait()
    for cp in gather_copies(i, q, b):
      cp.start()
    if prefetch:
      idx_copy(i + 2, (q + 2) % 4).start()
    pb = 1 - b
    for cp in gather_copies(i - 1, (q - 1) % 4, pb):
      cp.wait()
    out_copy(i - 1, pb).start()

  def body(t, carry):
    for j in range(4):
      step(2 + 4 * t + j, (2 + j) % 4, j % 2, True)
    return carry

  lax.fori_loop(0, (NCHUNK - 4) // 4, body, 0)

  step(NCHUNK - 2, (NCHUNK - 2) % 4, 0, False)
  step(NCHUNK - 1, (NCHUNK - 1) % 4, 1, False)
  for cp in gather_copies(NCHUNK - 1, (NCHUNK - 1) % 4, 1):
    cp.wait()
  out_copy(NCHUNK - 1, 1).start()
  out_copy(NCHUNK - 2, 0).wait()
  out_copy(NCHUNK - 1, 1).wait()


def kernel(week_numbers, week_embed):
  idx = week_numbers.astype(jnp.int32).reshape(NROWS, CH)
  out = _emb_lookup(idx, week_embed)
  return out.reshape(BATCH, HIST, HIDDEN)


# speedup vs baseline: 1.2088x; 1.2088x over previous
import functools

import jax
import jax.numpy as jnp
from jax import lax
from jax.experimental import pallas as pl
from jax.experimental.pallas import tpu as pltpu
from jax.experimental.pallas import tpu_sc as plsc

BATCH = 16384
HIST = 200
HIDDEN = 64
TABLE_ROWS = 1000

NC, NS = 2, 16
NW = NC * NS
B = BATCH * HIST
IDX_PER_W = B // NW
CH = 512
NCHUNK = IDX_PER_W // CH
NROWS = B // CH
GATHER_SPLITS = [(0, 128), (128, 128), (256, 128), (384, 128)]

_mesh = plsc.VectorSubcoreMesh(core_axis_name="c", subcore_axis_name="s")


@functools.partial(
    pl.kernel,
    out_type=jax.ShapeDtypeStruct((B, HIDDEN), jnp.float32),
    mesh=_mesh,
    scratch_types=[
        pltpu.VMEM((4, 1, CH), jnp.int32),
        pltpu.VMEM((2, CH, HIDDEN), jnp.float32),
        pltpu.VMEM_SHARED((TABLE_ROWS, HIDDEN), jnp.float32),
        pltpu.SemaphoreType.DMA,
        pltpu.SemaphoreType.DMA,
        pltpu.SemaphoreType.DMA,
        pltpu.SemaphoreType.DMA,
        pltpu.SemaphoreType.DMA,
        pltpu.SemaphoreType.DMA,
        pltpu.SemaphoreType.DMA,
        pltpu.SemaphoreType.DMA,
    ],
    compiler_params=pltpu.CompilerParams(use_tc_tiling_on_sc=False),
)
def _emb_lookup(idx_hbm, table_hbm, out_hbm, idx_v, rows_v, table_s,
                is0, is1, is2, is3, gs0, gs1, os0, os1):
  isems = (is0, is1, is2, is3)
  gsems = (gs0, gs1)
  osems = (os0, os1)
  sid = lax.axis_index("s")
  wid = sid * NC + lax.axis_index("c")
  wbase = wid * IDX_PER_W
  wrow = wid * NCHUNK

  @pl.when(sid == 0)
  def _():
    pltpu.sync_copy(table_hbm, table_s)

  plsc.subcore_barrier()

  def idx_copy(i, q):
    return pltpu.make_async_copy(
        idx_hbm.at[pl.ds(wrow + i, 1)], idx_v.at[q], isems[q])

  def gather_copies(i, q, b):
    src = table_s
    return [
        pltpu.make_async_copy(
            src.at[idx_v.at[q, 0, pl.ds(o, n)]],
            rows_v.at[b, pl.ds(o, n)], gsems[b])
        for o, n in GATHER_SPLITS
    ]

  def out_copy(i, b):
    return pltpu.make_async_copy(
        rows_v.at[b],
        out_hbm.at[pl.ds(wbase + i * CH, CH)], osems[b])

  idx_copy(0, 0).start()
  idx_copy(1, 1).start()

  idx_copy(0, 0).wait()
  for cp in gather_copies(0, 0, 0):
    cp.start()
  idx_copy(2, 2).start()

  idx_copy(1, 1).wait()
  for cp in gather_copies(1, 1, 1):
    cp.start()
  idx_copy(3, 3).start()
  for cp in gather_copies(0, 0, 0):
    cp.wait()
  out_copy(0, 0).start()

  def step(i, q, b, prefetch):
    idx_copy(i, q).wait()
    out_copy(i - 2, b).wait()
    for cp in gather_copies(i, q, b):
      cp.start()
    if prefetch:
      idx_copy(i + 2, (q + 2) % 4).start()
    pb = 1 - b
    for cp in gather_copies(i - 1, (q - 1) % 4, pb):
      cp.wait()
    out_copy(i - 1, pb).start()

  def body(t, carry):
    for j in range(4):
      step(2 + 4 * t + j, (2 + j) % 4, j % 2, True)
    return carry

  lax.fori_loop(0, (NCHUNK - 4) // 4, body, 0)

  step(NCHUNK - 2, (NCHUNK - 2) % 4, 0, False)
  step(NCHUNK - 1, (NCHUNK - 1) % 4, 1, False)
  for cp in gather_copies(NCHUNK - 1, (NCHUNK - 1) % 4, 1):
    cp.wait()
  out_copy(NCHUNK - 1, 1).start()
  out_copy(NCHUNK - 2, 0).wait()
  out_copy(NCHUNK - 1, 1).wait()


def kernel(week_numbers, week_embed):
  idx = week_numbers.astype(jnp.int32).reshape(NROWS, CH)
  out = _emb_lookup(idx, week_embed)
  return out.reshape(BATCH, HIST, HIDDEN)


# speedup vs baseline: 1.2108x; 1.0016x over previous
import functools

import jax
import jax.numpy as jnp
from jax import lax
from jax.experimental import pallas as pl
from jax.experimental.pallas import tpu as pltpu
from jax.experimental.pallas import tpu_sc as plsc

BATCH = 16384
HIST = 200
HIDDEN = 64
TABLE_ROWS = 1000

NC, NS = 2, 16
NW = NC * NS
B = BATCH * HIST
IDX_PER_W = B // NW
CH = 512
NCHUNK = IDX_PER_W // CH
NROWS = B // CH
GATHER_SPLITS = [(0, 128), (128, 128), (256, 128), (384, 128)]

_mesh = plsc.VectorSubcoreMesh(core_axis_name="c", subcore_axis_name="s")


@functools.partial(
    pl.kernel,
    out_type=jax.ShapeDtypeStruct((B, HIDDEN), jnp.float32),
    mesh=_mesh,
    scratch_types=[
        pltpu.VMEM((4, 1, CH), jnp.int32),
        pltpu.VMEM((2, CH, HIDDEN), jnp.float32),
        pltpu.VMEM_SHARED((TABLE_ROWS, HIDDEN), jnp.float32),
        pltpu.SemaphoreType.DMA,
        pltpu.SemaphoreType.DMA,
        pltpu.SemaphoreType.DMA,
        pltpu.SemaphoreType.DMA,
        pltpu.SemaphoreType.DMA,
        pltpu.SemaphoreType.DMA,
        pltpu.SemaphoreType.DMA,
        pltpu.SemaphoreType.DMA,
    ],
    compiler_params=pltpu.CompilerParams(use_tc_tiling_on_sc=False),
)
def _emb_lookup(idx_hbm, table_hbm, out_hbm, idx_v, rows_v, table_s,
                is0, is1, is2, is3, gs0, gs1, os0, os1):
  isems = (is0, is1, is2, is3)
  gsems = (gs0, gs1)
  osems = (os0, os1)
  sid = lax.axis_index("s")
  wid = sid * NC + lax.axis_index("c")
  wbase = wid * IDX_PER_W
  wrow = wid * NCHUNK

  @pl.when(sid == 0)
  def _():
    pltpu.sync_copy(table_hbm, table_s)

  plsc.subcore_barrier()

  def idx_copy(i, q):
    return pltpu.make_async_copy(
        idx_hbm.at[pl.ds(wrow + i, 1)], idx_v.at[q], isems[q])

  def gather_copies(i, q, b):
    return [
        pltpu.make_async_copy(
            table_s.at[idx_v.at[q, 0, pl.ds(o, n)]],
            rows_v.at[b, pl.ds(o, n)], gsems[b])
        for o, n in GATHER_SPLITS
    ]

  def out_copy(i, b):
    return pltpu.make_async_copy(
        rows_v.at[b],
        out_hbm.at[pl.ds(wbase + i * CH, CH)], osems[b])

  idx_copy(0, 0).start()
  idx_copy(1, 1).start()

  idx_copy(0, 0).wait()
  for cp in gather_copies(0, 0, 0):
    cp.start()
  idx_copy(2, 2).start()

  idx_copy(1, 1).wait()
  for cp in gather_copies(1, 1, 1):
    cp.start()
  idx_copy(3, 3).start()
  for cp in gather_copies(0, 0, 0):
    cp.wait()
  out_copy(0, 0).start()

  def step(i, q, b, prefetch):
    idx_copy(i, q).wait()
    out_copy(i - 2, b).wait()
    for cp in gather_copies(i, q, b):
      cp.start()
    if prefetch:
      idx_copy(i + 2, (q + 2) % 4).start()
    pb = 1 - b
    for cp in gather_copies(i - 1, (q - 1) % 4, pb):
      cp.wait()
    out_copy(i - 1, pb).start()

  def body(t, carry):
    for j in range(4):
      step(2 + 4 * t + j, (2 + j) % 4, j % 2, True)
    return carry

  lax.fori_loop(0, (NCHUNK - 4) // 4, body, 0)

  step(NCHUNK - 2, (NCHUNK - 2) % 4, 0, False)
  step(NCHUNK - 1, (NCHUNK - 1) % 4, 1, False)
  for cp in gather_copies(NCHUNK - 1, (NCHUNK - 1) % 4, 1):
    cp.wait()
  out_copy(NCHUNK - 1, 1).start()
  out_copy(NCHUNK - 2, 0).wait()
  out_copy(NCHUNK - 1, 1).wait()


def kernel(week_numbers, week_embed):
  idx = week_numbers.astype(jnp.int32).reshape(NROWS, CH)
  out = _emb_lookup(idx, week_embed)
  return out.reshape(BATCH, HIST, HIDDEN)
